# baseline (speedup 1.0000x reference)
import jax
import jax.numpy as jnp
from jax import lax
from jax.experimental import pallas as pl
from jax.experimental.pallas import tpu as pltpu
from jax.experimental.pallas import tpu_sc as plsc

N = 10000
E = 320000
D = 128

NC = 2
NS = 16
NW = NC * NS
CH = 128
CPW = 80
ECH = NW * CPW
PHC = 40
NB = 2
NTRASH = 8
NA = N + NTRASH
RPT = 624
RTAIL = NA - NS * RPT
OTAIL = N - NS * RPT

_MESH = plsc.VectorSubcoreMesh(core_axis_name="c", subcore_axis_name="s")
_PARAMS = pltpu.CompilerParams(needs_layout_passes=False)


def _sc_xlate_body(cids, src2, out, cids_v, buf):
    wid = lax.axis_index("c") * NS + lax.axis_index("s")
    pltpu.sync_copy(cids, cids_v)
    pltpu.sync_copy(src2.at[pl.ds(wid * CPW, CPW)], buf)

    def xlate(i, carry):
        r = i // (CH // 16)
        g = i % (CH // 16)
        v = buf[r, pl.ds(g * 16, 16)]
        buf[r, pl.ds(g * 16, 16)] = plsc.load_gather(cids_v, [v])
        return carry

    lax.fori_loop(0, CPW * (CH // 16), xlate, 0)
    pltpu.sync_copy(buf, out.at[pl.ds(wid * CPW, CPW)])


def _sc_xlate(cids, src2):
    kern = pl.kernel(
        _sc_xlate_body,
        out_type=jax.ShapeDtypeStruct((ECH, CH), jnp.int32),
        mesh=_MESH,
        compiler_params=_PARAMS,
        scratch_types=[
            pltpu.VMEM((N,), jnp.int32),
            pltpu.VMEM((CPW, CH), jnp.int32),
        ],
    )
    return kern(cids, src2)


def _sc_agg_body(table, sidx2, didx2, zin, out,
                 sidx, didx, rows0, rows1, acc, sem0, sem1, ssem0, ssem1):
    c = lax.axis_index("c")
    s = lax.axis_index("s")
    wid = c * NS + s
    rows = (rows0, rows1)
    sems = (sem0, sem1)
    ssems = (ssem0, ssem1)

    pltpu.sync_copy(zin.at[pl.ds(s * RPT, RPT)], acc.at[pl.ds(s * RPT, RPT)])

    @pl.when(s == 0)
    def _():
        pltpu.sync_copy(zin.at[pl.ds(NS * RPT, RTAIL)],
                        acc.at[pl.ds(NS * RPT, RTAIL)])

    plsc.subcore_barrier()

    for p in range(CPW // PHC):
        base = wid * CPW + p * PHC
        pltpu.sync_copy(sidx2.at[pl.ds(base, PHC)], sidx)
        pltpu.sync_copy(didx2.at[pl.ds(base, PHC)], didx)

        for b in range(NB):
            pltpu.async_copy(table.at[sidx.at[b]], rows[b], sems[b])

        def pipe(i, carry):
            for b in range(NB):
                j = i * NB + b
                bp = (b + NB - 1) % NB
                pltpu.make_async_copy(
                    table.at[sidx.at[j]], rows[b], sems[b]).wait()
                pltpu.async_copy(rows[b], acc.at[didx.at[j]], ssems[b],
                                 add=True)

                @pl.when(j > 0)
                def _():
                    pltpu.make_async_copy(
                        rows[bp], acc.at[didx.at[j - 1]], ssems[bp]).wait()

                @pl.when((0 < j) & (j + 1 < PHC))
                def _():
                    pltpu.async_copy(
                        table.at[sidx.at[j + 1]], rows[bp], sems[bp])
            return carry

        lax.fori_loop(0, PHC // NB, pipe, 0)
        pltpu.make_async_copy(
            rows[(PHC - 1) % NB], acc.at[didx.at[PHC - 1]],
            ssems[(PHC - 1) % NB]).wait()

    plsc.subcore_barrier()
    pltpu.sync_copy(acc.at[pl.ds(s * RPT, RPT)],
                    out.at[c, pl.ds(s * RPT, RPT)])

    @pl.when(s == 0)
    def _():
        pltpu.sync_copy(acc.at[pl.ds(NS * RPT, OTAIL)],
                        out.at[c, pl.ds(NS * RPT, OTAIL)])


def _sc_agg(table, sidx2, didx2, zin):
    kern = pl.kernel(
        _sc_agg_body,
        out_type=jax.ShapeDtypeStruct((NC, N, D), jnp.float32),
        mesh=_MESH,
        compiler_params=_PARAMS,
        scratch_types=[
            pltpu.VMEM((PHC, CH), jnp.int32),
            pltpu.VMEM((PHC, CH), jnp.int32),
            pltpu.VMEM((CH, D), jnp.float32),
            pltpu.VMEM((CH, D), jnp.float32),
            pltpu.VMEM_SHARED((NA, D), jnp.float32),
            pltpu.SemaphoreType.DMA,
            pltpu.SemaphoreType.DMA,
            pltpu.SemaphoreType.DMA,
            pltpu.SemaphoreType.DMA,
        ],
    )
    return kern(table, sidx2, didx2, zin)


def _tc_mlp_body(p_ref, w_ref, b_ref, o_ref):
    x = p_ref[0] + p_ref[1]
    y = jnp.dot(x, w_ref[...], preferred_element_type=jnp.float32) + b_ref[...]
    o_ref[...] = jnp.maximum(y, 0.0)


def _tc_mlp(partials, w, b):
    br = 2000
    return pl.pallas_call(
        _tc_mlp_body,
        grid=(N // br,),
        in_specs=[
            pl.BlockSpec((NC, br, D), lambda i: (0, i, 0)),
            pl.BlockSpec((D, D), lambda i: (0, 0)),
            pl.BlockSpec((1, D), lambda i: (0, 0)),
        ],
        out_specs=pl.BlockSpec((br, D), lambda i: (i, 0)),
        out_shape=jax.ShapeDtypeStruct((N, D), jnp.float32),
    )(partials, w, b.reshape(1, D))


def kernel(cncpt_ids, edge_index, concept_table, W1, b1, W2, b2):
    cids = cncpt_ids.astype(jnp.int32)
    src = edge_index[0].astype(jnp.int32)
    dst = edge_index[1].astype(jnp.int32)
    npad = ECH * CH - E
    ar = jnp.arange(npad, dtype=jnp.int32)
    src2 = jnp.concatenate([src, ar % N]).reshape(ECH, CH)
    dst2 = jnp.concatenate([dst, N + (ar % NTRASH)]).reshape(ECH, CH)
    zin = jnp.zeros((NA, D), jnp.float32)
    idx2 = _sc_xlate(cids, src2)
    p1 = _sc_agg(concept_table, idx2, dst2, zin)
    h = _tc_mlp(p1, W1, b1)
    p2 = _sc_agg(h, src2, dst2, zin)
    return _tc_mlp(p2, W2, b2)


# speedup vs baseline: 1.1570x; 1.1570x over previous
import functools

import jax
import jax.numpy as jnp
from jax import lax
from jax.experimental import pallas as pl
from jax.experimental.pallas import tpu as pltpu
from jax.experimental.pallas import tpu_sc as plsc

N = 10000
E = 320000
D = 128

NC = 2
NS = 16
NW = NC * NS
CH = 128
CPW = 80
ECH = NW * CPW
NB = 2
NTRASH = 8
NA = N + NTRASH
RPT = 624
RTAIL = NA - NS * RPT
OTAIL = N - NS * RPT

_MESH = plsc.VectorSubcoreMesh(core_axis_name="c", subcore_axis_name="s")
_PARAMS = pltpu.CompilerParams(needs_layout_passes=False)


def _sc_agg_body(phc, xlate, table, cids, sidx2, didx2, zin, out,
                 cids_v, sidx, didx, rows0, rows1, acc, sem0, sem1):
    c = lax.axis_index("c")
    s = lax.axis_index("s")
    wid = c * NS + s
    rows = (rows0, rows1)
    sems = (sem0, sem1)

    pltpu.sync_copy(zin.at[pl.ds(s * RPT, RPT)], acc.at[pl.ds(s * RPT, RPT)])

    @pl.when(s == 0)
    def _():
        pltpu.sync_copy(zin.at[pl.ds(NS * RPT, RTAIL)],
                        acc.at[pl.ds(NS * RPT, RTAIL)])

    if xlate:
        pltpu.sync_copy(cids, cids_v)

    plsc.subcore_barrier()

    def translate(jj):
        if xlate:
            for g in range(CH // 16):
                v = sidx[jj, pl.ds(g * 16, 16)]
                sidx[jj, pl.ds(g * 16, 16)] = plsc.load_gather(cids_v, [v])

    for p in range(CPW // phc):
        base = wid * CPW + p * phc
        pltpu.sync_copy(sidx2.at[pl.ds(base, phc)], sidx)
        pltpu.sync_copy(didx2.at[pl.ds(base, phc)], didx)

        for b in range(NB):
            translate(b)
            pltpu.async_copy(table.at[sidx.at[b]], rows[b], sems[b])

        def pipe(i, carry):
            for b in range(NB):
                j = i * NB + b
                pltpu.make_async_copy(
                    table.at[sidx.at[j]], rows[b], sems[b]).wait()
                pltpu.sync_copy(rows[b], acc.at[didx.at[j]], add=True)

                @pl.when(j + NB < phc)
                def _():
                    translate(j + NB)
                    pltpu.async_copy(
                        table.at[sidx.at[j + NB]], rows[b], sems[b])
            return carry

        lax.fori_loop(0, phc // NB, pipe, 0)

    plsc.subcore_barrier()
    pltpu.sync_copy(acc.at[pl.ds(s * RPT, RPT)],
                    out.at[c, pl.ds(s * RPT, RPT)])

    @pl.when(s == 0)
    def _():
        pltpu.sync_copy(acc.at[pl.ds(NS * RPT, OTAIL)],
                        out.at[c, pl.ds(NS * RPT, OTAIL)])


def _sc_agg(table, cids, sidx2, didx2, zin, xlate):
    phc = 16 if xlate else 40
    kern = pl.kernel(
        functools.partial(_sc_agg_body, phc, xlate),
        out_type=jax.ShapeDtypeStruct((NC, N, D), jnp.float32),
        mesh=_MESH,
        compiler_params=_PARAMS,
        scratch_types=[
            pltpu.VMEM((N if xlate else 8,), jnp.int32),
            pltpu.VMEM((phc, CH), jnp.int32),
            pltpu.VMEM((phc, CH), jnp.int32),
            pltpu.VMEM((CH, D), jnp.float32),
            pltpu.VMEM((CH, D), jnp.float32),
            pltpu.VMEM_SHARED((NA, D), jnp.float32),
            pltpu.SemaphoreType.DMA,
            pltpu.SemaphoreType.DMA,
        ],
    )
    return kern(table, cids, sidx2, didx2, zin)


def _tc_mlp_body(p_ref, w_ref, b_ref, o_ref):
    x = p_ref[0] + p_ref[1]
    y = jnp.dot(x, w_ref[...], preferred_element_type=jnp.float32) + b_ref[...]
    o_ref[...] = jnp.maximum(y, 0.0)


def _tc_mlp(partials, w, b):
    br = 2000
    return pl.pallas_call(
        _tc_mlp_body,
        grid=(N // br,),
        in_specs=[
            pl.BlockSpec((NC, br, D), lambda i: (0, i, 0)),
            pl.BlockSpec((D, D), lambda i: (0, 0)),
            pl.BlockSpec((1, D), lambda i: (0, 0)),
        ],
        out_specs=pl.BlockSpec((br, D), lambda i: (i, 0)),
        out_shape=jax.ShapeDtypeStruct((N, D), jnp.float32),
    )(partials, w, b.reshape(1, D))


def kernel(cncpt_ids, edge_index, concept_table, W1, b1, W2, b2):
    cids = cncpt_ids.astype(jnp.int32)
    src = edge_index[0].astype(jnp.int32)
    dst = edge_index[1].astype(jnp.int32)
    npad = ECH * CH - E
    ar = jnp.arange(npad, dtype=jnp.int32)
    src2 = jnp.concatenate([src, ar % N]).reshape(ECH, CH)
    dst2 = jnp.concatenate([dst, N + (ar % NTRASH)]).reshape(ECH, CH)
    zin = jnp.zeros((NA, D), jnp.float32)
    p1 = _sc_agg(concept_table, cids, src2, dst2, zin, True)
    h = _tc_mlp(p1, W1, b1)
    p2 = _sc_agg(h, cids, src2, dst2, zin, False)
    return _tc_mlp(p2, W2, b2)


# speedup vs baseline: 1.1695x; 1.0108x over previous
import functools

import jax
import jax.numpy as jnp
from jax import lax
from jax.experimental import pallas as pl
from jax.experimental.pallas import tpu as pltpu
from jax.experimental.pallas import tpu_sc as plsc

N = 10000
E = 320000
D = 128

NC = 2
NS = 16
NW = NC * NS
CH = 128
CPW = 80
ECH = NW * CPW
NB = 2
NTRASH = 8
NA = N + NTRASH
RPT = 624
RTAIL = NA - NS * RPT
OTAIL = N - NS * RPT

_MESH = plsc.VectorSubcoreMesh(core_axis_name="c", subcore_axis_name="s")
_PARAMS = pltpu.CompilerParams(needs_layout_passes=False)


def _sc_agg_body(phases, xlate, table, cids, sidx2, didx2, zin, out,
                 cids_v, sidx, didx, rows0, rows1, acc, sem0, sem1):
    c = lax.axis_index("c")
    s = lax.axis_index("s")
    wid = c * NS + s
    rows = (rows0, rows1)
    sems = (sem0, sem1)

    pltpu.sync_copy(zin, acc.at[pl.ds(s * RPT, RPT)])

    @pl.when(s == 0)
    def _():
        pltpu.sync_copy(zin.at[pl.ds(0, RTAIL)],
                        acc.at[pl.ds(NS * RPT, RTAIL)])

    if xlate:
        pltpu.sync_copy(cids, cids_v)

    plsc.subcore_barrier()

    def translate(jj):
        if xlate:
            for g in range(CH // 16):
                v = sidx[jj, pl.ds(g * 16, 16)]
                sidx[jj, pl.ds(g * 16, 16)] = plsc.load_gather(cids_v, [v])

    off = 0
    for phc in phases:
        base = wid * CPW + off
        off += phc
        pltpu.sync_copy(sidx2.at[pl.ds(base, phc)], sidx.at[pl.ds(0, phc)])
        pltpu.sync_copy(didx2.at[pl.ds(base, phc)], didx.at[pl.ds(0, phc)])

        for b in range(NB):
            translate(b)
            pltpu.async_copy(table.at[sidx.at[b]], rows[b], sems[b])

        def pipe(i, carry):
            for b in range(NB):
                j = i * NB + b
                pltpu.make_async_copy(
                    table.at[sidx.at[j]], rows[b], sems[b]).wait()
                pltpu.sync_copy(rows[b], acc.at[didx.at[j]], add=True)

                @pl.when(j + NB < phc)
                def _():
                    translate(j + NB)
                    pltpu.async_copy(
                        table.at[sidx.at[j + NB]], rows[b], sems[b])
            return carry

        lax.fori_loop(0, phc // NB, pipe, 0)

    plsc.subcore_barrier()
    pltpu.sync_copy(acc.at[pl.ds(s * RPT, RPT)],
                    out.at[c, pl.ds(s * RPT, RPT)])

    @pl.when(s == 0)
    def _():
        pltpu.sync_copy(acc.at[pl.ds(NS * RPT, OTAIL)],
                        out.at[c, pl.ds(NS * RPT, OTAIL)])


def _sc_agg(table, cids, sidx2, didx2, zin, xlate):
    phases = (24, 24, 24, 8) if xlate else (40, 40)
    kern = pl.kernel(
        functools.partial(_sc_agg_body, phases, xlate),
        out_type=jax.ShapeDtypeStruct((NC, N, D), jnp.float32),
        mesh=_MESH,
        compiler_params=_PARAMS,
        scratch_types=[
            pltpu.VMEM((N if xlate else 8,), jnp.int32),
            pltpu.VMEM((max(phases), CH), jnp.int32),
            pltpu.VMEM((max(phases), CH), jnp.int32),
            pltpu.VMEM((CH, D), jnp.float32),
            pltpu.VMEM((CH, D), jnp.float32),
            pltpu.VMEM_SHARED((NA, D), jnp.float32),
            pltpu.SemaphoreType.DMA,
            pltpu.SemaphoreType.DMA,
        ],
    )
    return kern(table, cids, sidx2, didx2, zin)


def _tc_mlp_body(p_ref, w_ref, b_ref, o_ref):
    x = p_ref[0] + p_ref[1]
    y = jnp.dot(x, w_ref[...], preferred_element_type=jnp.float32) + b_ref[...]
    o_ref[...] = jnp.maximum(y, 0.0)


def _tc_mlp(partials, w, b):
    br = 2000
    return pl.pallas_call(
        _tc_mlp_body,
        grid=(N // br,),
        in_specs=[
            pl.BlockSpec((NC, br, D), lambda i: (0, i, 0)),
            pl.BlockSpec((D, D), lambda i: (0, 0)),
            pl.BlockSpec((1, D), lambda i: (0, 0)),
        ],
        out_specs=pl.BlockSpec((br, D), lambda i: (i, 0)),
        out_shape=jax.ShapeDtypeStruct((N, D), jnp.float32),
    )(partials, w, b.reshape(1, D))


def kernel(cncpt_ids, edge_index, concept_table, W1, b1, W2, b2):
    cids = cncpt_ids.astype(jnp.int32)
    src = edge_index[0].astype(jnp.int32)
    dst = edge_index[1].astype(jnp.int32)
    npad = ECH * CH - E
    ar = jnp.arange(npad, dtype=jnp.int32)
    src2 = jnp.concatenate([src, ar % N]).reshape(ECH, CH)
    dst2 = jnp.concatenate([dst, N + (ar % NTRASH)]).reshape(ECH, CH)
    zin = jnp.zeros((RPT, D), jnp.float32)
    p1 = _sc_agg(concept_table, cids, src2, dst2, zin, True)
    h = _tc_mlp(p1, W1, b1)
    p2 = _sc_agg(h, cids, src2, dst2, zin, False)
    return _tc_mlp(p2, W2, b2)
